# Initial kernel scaffold; baseline (speedup 1.0000x reference)
import jax
import jax.numpy as jnp
from jax.experimental import pallas as pl


def kernel(logits, labels):
    raise NotImplementedError("write your pallas kernel here")


# speedup vs baseline: 20.1610x; 20.1610x over previous
import functools

import jax
import jax.numpy as jnp
from jax import lax
from jax.experimental import pallas as pl
from jax.experimental.pallas import tpu as pltpu
from jax.experimental.pallas import tpu_sc as plsc

N_ROWS = 64
M_COLS = 100000
CHUNK = 4000
N_CHUNKS = M_COLS // CHUNK
VPC = CHUNK // 16
KEY_CAP = M_COLS + 16
DELTA = 5.0
INV_R = 100
MININT = -(2 ** 31)

_MESH = plsc.VectorSubcoreMesh(core_axis_name="c", subcore_axis_name="s")


@functools.partial(
    pl.kernel,
    mesh=_MESH,
    out_type=jax.ShapeDtypeStruct((512,), jnp.float32),
    compiler_params=pltpu.CompilerParams(needs_layout_passes=False),
    scratch_types=[
        pltpu.VMEM((CHUNK,), jnp.float32),
        pltpu.VMEM((CHUNK,), jnp.int32),
        pltpu.VMEM((KEY_CAP,), jnp.int32),
        pltpu.VMEM((256,), jnp.int32),
        pltpu.VMEM((256,), jnp.float32),
        pltpu.VMEM((16,), jnp.float32),
    ],
)
def _mmcl_sc(logits_hbm, labels_hbm, out_hbm, vbuf, lbuf, keys, hcnt, hsum,
             stage):
    wid = lax.axis_index("s") * 2 + lax.axis_index("c")
    iota = lax.iota(jnp.int32, 16)
    zeros_i = jnp.zeros((16,), jnp.int32)
    zeros_f = jnp.zeros((16,), jnp.float32)
    ones_i = jnp.ones((16,), jnp.int32)

    def zero_hists(g, carry):
        sl = pl.ds(g * 16, 16)
        hcnt[sl] = zeros_i
        hsum[sl] = zeros_f
        return carry

    def select_bin(k_rem):
        def sel_a(gi, carry):
            bcount, runs = carry
            g = 15 - gi
            c = hcnt[pl.ds(g * 16, 16)]
            incl = plsc.cumsum(lax.rev(c, (0,)))
            suff = lax.rev(incl, (0,)) + runs
            m = suff >= k_rem
            bcount = bcount + jnp.sum(jnp.where(m, ones_i, 0))
            runs = runs + jnp.sum(c)
            return bcount, runs

        bcount, _ = lax.fori_loop(0, 16, sel_a, (jnp.int32(0), jnp.int32(0)))
        b = bcount - 1

        def sel_b(g, carry):
            ca, sa = carry
            idxv = g * 16 + iota
            m = idxv > b
            ca = ca + jnp.where(m, hcnt[pl.ds(g * 16, 16)], 0)
            sa = sa + jnp.where(m, hsum[pl.ds(g * 16, 16)], 0.0)
            return ca, sa

        ca, sa = lax.fori_loop(0, 16, sel_b, (zeros_i, zeros_f))
        return b, jnp.sum(ca), jnp.sum(sa)

    loss_acc = zeros_f
    for r in range(2):
        row = wid * 2 + r
        lax.fori_loop(0, 16, zero_hists, 0)

        def chunk_body(c, carry):
            possum, npos, offv = carry
            off_el = pl.multiple_of(row * M_COLS + c * CHUNK, CHUNK)
            pltpu.sync_copy(logits_hbm.at[pl.ds(off_el, CHUNK)], vbuf)
            pltpu.sync_copy(labels_hbm.at[pl.ds(off_el, CHUNK)], lbuf)

            def vec_body(i, icarry):
                possum, npos, offv = icarry
                sl = pl.ds(i * 16, 16)
                v = vbuf[sl]
                labv = lbuf[sl]
                isneg = labv == 0
                ispos = jnp.logical_not(isneg)
                d = 1.0 - v
                possum = possum + jnp.where(ispos, d * d, 0.0)
                npos = npos + jnp.where(ispos, ones_i, 0)
                bbits = lax.bitcast_convert_type(v, jnp.int32)
                sgn = jnp.right_shift(bbits, 31)
                key = jnp.bitwise_xor(bbits, jnp.bitwise_or(sgn, jnp.int32(MININT)))
                incl = plsc.cumsum(jnp.where(isneg, ones_i, 0))
                posn = offv + incl - 1
                plsc.store_scatter(keys, [posn], key, mask=isneg)
                bin0 = jnp.bitwise_and(jnp.right_shift(key, 24), 255)
                plsc.addupdate_scatter(hcnt, [bin0], ones_i, mask=isneg)
                e = 1.0 + v
                plsc.addupdate_scatter(hsum, [bin0], e * e, mask=isneg)
                cntv = plsc.all_reduce_population_count(isneg)
                return possum, npos, offv + cntv

            return lax.fori_loop(0, VPC, vec_body, (possum, npos, offv))

        possum, npos, offv = lax.fori_loop(
            0, N_CHUNKS, chunk_body, (zeros_f, zeros_i, zeros_i))

        n_pos = jnp.sum(npos)
        pos_sum = jnp.sum(possum)
        n_neg = jnp.max(offv)
        num = n_neg // INV_R

        k_rem = num
        prefix = jnp.int32(0)
        sum_gt = jnp.float32(0.0)
        niter = (n_neg + 15) // 16
        for p in range(4):
            if p > 0:
                sh_hi = 32 - 8 * p
                hi_mask = (1 << (32 - sh_hi)) - 1
                sh = 24 - 8 * p
                lax.fori_loop(0, 16, zero_hists, 0)

                def scan_body(i, carry, sh_hi=sh_hi, hi_mask=hi_mask, sh=sh,
                              prefix=prefix):
                    sl = pl.ds(i * 16, 16)
                    key = keys[sl]
                    lanev = i * 16 + iota
                    valid = lanev < n_neg
                    top = jnp.bitwise_and(jnp.right_shift(key, sh_hi), hi_mask)
                    matched = jnp.logical_and(valid, top == prefix)
                    binv = jnp.bitwise_and(jnp.right_shift(key, sh), 255)
                    negk = key >= 0
                    borig = jnp.where(negk, jnp.bitwise_not(key),
                                      jnp.bitwise_xor(key, jnp.int32(MININT)))
                    vdec = lax.bitcast_convert_type(borig, jnp.float32)
                    e = 1.0 + vdec
                    plsc.addupdate_scatter(hcnt, [binv], ones_i, mask=matched)
                    plsc.addupdate_scatter(hsum, [binv], e * e, mask=matched)
                    return carry

                lax.fori_loop(0, niter, scan_body, 0)

            b, cnt_above, sum_above = select_bin(k_rem)
            k_rem = k_rem - cnt_above
            sum_gt = sum_gt + sum_above
            prefix = jnp.bitwise_or(jnp.left_shift(prefix, 8), b)

        tkv = jnp.full((16,), 1, jnp.int32) * prefix
        negk = tkv >= 0
        borig = jnp.where(negk, jnp.bitwise_not(tkv),
                          jnp.bitwise_xor(tkv, jnp.int32(MININT)))
        vtv = lax.bitcast_convert_type(borig, jnp.float32)
        ev = 1.0 + vtv
        kremf = lax.convert_element_type(k_rem, jnp.float32)
        tie_v = jnp.where(jnp.logical_and(iota == 0, k_rem > 0),
                          kremf * ev * ev, 0.0)
        hard_sum = sum_gt + jnp.sum(tie_v)

        ones_f = jnp.full((16,), 1.0, jnp.float32)
        nposf_v = ones_f * lax.convert_element_type(n_pos, jnp.float32)
        numf_v = ones_f * lax.convert_element_type(num, jnp.float32)
        loss_v = (jnp.float32(DELTA) * (ones_f * pos_sum) / nposf_v
                  + (ones_f * hard_sum) / numf_v)
        loss_acc = loss_acc + loss_v

    stage[...] = jnp.where(iota == 0, loss_acc, 0.0)
    out_off = pl.multiple_of(wid * 16, 16)
    pltpu.sync_copy(stage, out_hbm.at[pl.ds(out_off, 16)])


def kernel(logits, labels):
    out = _mmcl_sc(logits.reshape(-1), labels.reshape(-1))
    return jnp.sum(out) * jnp.float32(1.0 / N_ROWS)


# speedup vs baseline: 31.1001x; 1.5426x over previous
import functools

import jax
import jax.numpy as jnp
from jax import lax
from jax.experimental import pallas as pl
from jax.experimental.pallas import tpu as pltpu
from jax.experimental.pallas import tpu_sc as plsc

N_ROWS = 64
M_COLS = 100000
CH = 2000
NDC = M_COLS // (2 * CH)
VPC = CH // 16
KEY_CAP = M_COLS + 16
DELTA = 5.0
INV_R = 100
MININT = -(2 ** 31)

_MESH = plsc.VectorSubcoreMesh(core_axis_name="c", subcore_axis_name="s")


@functools.partial(
    pl.kernel,
    mesh=_MESH,
    out_type=jax.ShapeDtypeStruct((512,), jnp.float32),
    compiler_params=pltpu.CompilerParams(needs_layout_passes=False),
    scratch_types=[
        pltpu.VMEM((CH,), jnp.float32),
        pltpu.VMEM((CH,), jnp.float32),
        pltpu.VMEM((CH,), jnp.int32),
        pltpu.VMEM((CH,), jnp.int32),
        pltpu.VMEM((KEY_CAP,), jnp.int32),
        pltpu.VMEM((256,), jnp.int32),
        pltpu.VMEM((16,), jnp.float32),
        pltpu.SemaphoreType.DMA,
        pltpu.SemaphoreType.DMA,
        pltpu.SemaphoreType.DMA,
        pltpu.SemaphoreType.DMA,
    ],
)
def _mmcl_sc(logits_hbm, labels_hbm, out_hbm, vbuf0, vbuf1, lbuf0, lbuf1,
             keys, hcnt, stage, sv0, sv1, sl0, sl1):
    wid = lax.axis_index("s") * 2 + lax.axis_index("c")
    iota = lax.iota(jnp.int32, 16)
    zeros_i = jnp.zeros((16,), jnp.int32)
    zeros_f = jnp.zeros((16,), jnp.float32)
    ones_i = jnp.ones((16,), jnp.int32)

    def zero_hist(g, carry):
        hcnt[pl.ds(g * 16, 16)] = zeros_i
        return carry

    def start_pair(base, vb, lb, sv, sl):
        pltpu.make_async_copy(logits_hbm.at[pl.ds(base, CH)], vb, sv).start()
        pltpu.make_async_copy(labels_hbm.at[pl.ds(base, CH)], lb, sl).start()

    def wait_pair(base, vb, lb, sv, sl):
        pltpu.make_async_copy(logits_hbm.at[pl.ds(base, CH)], vb, sv).wait()
        pltpu.make_async_copy(labels_hbm.at[pl.ds(base, CH)], lb, sl).wait()

    def chunk_compute(vb, lb, possum, offv):
        def vec_body(i, icarry):
            possum, offv = icarry
            sl = pl.ds(i * 16, 16)
            v = vb[sl]
            labv = lb[sl]
            isneg = labv == 0
            ispos = jnp.logical_not(isneg)
            d = 1.0 - v
            possum = possum + jnp.where(ispos, d * d, 0.0)
            bbits = lax.bitcast_convert_type(v, jnp.int32)
            sgn = jnp.right_shift(bbits, 31)
            key = jnp.bitwise_xor(bbits,
                                  jnp.bitwise_or(sgn, jnp.int32(MININT)))
            incl = plsc.cumsum(jnp.where(isneg, ones_i, 0))
            posn = offv + incl - 1
            plsc.store_scatter(keys, [posn], key, mask=isneg)
            bin0 = jnp.bitwise_and(jnp.right_shift(key, 24), 255)
            plsc.addupdate_scatter(hcnt, [bin0], ones_i, mask=isneg)
            return possum, offv + plsc.all_reduce_population_count(isneg)

        return lax.fori_loop(0, VPC, vec_body, (possum, offv))

    def select_bin(k_rem):
        def sel_a(gi, carry):
            bcount, runs = carry
            g = 15 - gi
            c = hcnt[pl.ds(g * 16, 16)]
            incl = plsc.cumsum(lax.rev(c, (0,)))
            suff = lax.rev(incl, (0,)) + runs
            m = suff >= k_rem
            bcount = bcount + jnp.sum(jnp.where(m, ones_i, 0))
            runs = runs + jnp.sum(c)
            return bcount, runs

        bcount, _ = lax.fori_loop(0, 16, sel_a, (jnp.int32(0), jnp.int32(0)))
        b = bcount - 1

        def sel_b(g, ca):
            m = (g * 16 + iota) > b
            return ca + jnp.where(m, hcnt[pl.ds(g * 16, 16)], 0)

        ca = lax.fori_loop(0, 16, sel_b, zeros_i)
        return b, jnp.sum(ca)

    def refine_scan(n_cur, b_sel, sh, acc, with_hist):
        def body(i, c):
            off2, acc = c
            key = keys[pl.ds(i * 16, 16)]
            valid = (i * 16 + iota) < n_cur
            binv = jnp.bitwise_and(jnp.right_shift(key, sh), 255)
            above = jnp.logical_and(valid, binv > b_sel)
            matched = jnp.logical_and(valid, binv == b_sel)
            negk = key >= 0
            borig = jnp.where(negk, jnp.bitwise_not(key),
                              jnp.bitwise_xor(key, jnp.int32(MININT)))
            vdec = lax.bitcast_convert_type(borig, jnp.float32)
            e = 1.0 + vdec
            acc = acc + jnp.where(above, e * e, 0.0)
            incl = plsc.cumsum(jnp.where(matched, ones_i, 0))
            posn = off2 + incl - 1
            plsc.store_scatter(keys, [posn], key, mask=matched)
            if with_hist:
                nbin = jnp.bitwise_and(jnp.right_shift(key, sh - 8), 255)
                plsc.addupdate_scatter(hcnt, [nbin], ones_i, mask=matched)
            return off2 + plsc.all_reduce_population_count(matched), acc

        niter = (n_cur + 15) // 16
        off2, acc = lax.fori_loop(0, niter, body, (zeros_i, acc))
        return jnp.max(off2), acc

    loss_acc = zeros_f
    for r in range(2):
        row = wid * 2 + r
        base0 = pl.multiple_of(row * M_COLS, 16)
        lax.fori_loop(0, 16, zero_hist, 0)

        start_pair(base0, vbuf0, lbuf0, sv0, sl0)

        def jbody(j, carry, base0=base0):
            possum, offv = carry
            base = pl.multiple_of(base0 + j * 2 * CH, 16)
            start_pair(base + CH, vbuf1, lbuf1, sv1, sl1)
            wait_pair(base, vbuf0, lbuf0, sv0, sl0)
            possum, offv = chunk_compute(vbuf0, lbuf0, possum, offv)

            @pl.when(j < NDC - 1)
            def _():
                start_pair(base + 2 * CH, vbuf0, lbuf0, sv0, sl0)

            wait_pair(base + CH, vbuf1, lbuf1, sv1, sl1)
            possum, offv = chunk_compute(vbuf1, lbuf1, possum, offv)
            return possum, offv

        possum, offv = lax.fori_loop(0, NDC, jbody, (zeros_f, zeros_i))

        pos_sum = jnp.sum(possum)
        n_neg = jnp.max(offv)
        n_pos = M_COLS - n_neg
        num = n_neg // INV_R

        k_rem = num
        acc = zeros_f
        b0, ca = select_bin(k_rem)
        k_rem = k_rem - ca
        lax.fori_loop(0, 16, zero_hist, 0)
        n1, acc = refine_scan(n_neg, b0, 24, acc, True)

        b1, ca = select_bin(k_rem)
        k_rem = k_rem - ca
        lax.fori_loop(0, 16, zero_hist, 0)
        n2, acc = refine_scan(n1, b1, 16, acc, True)

        b2, ca = select_bin(k_rem)
        k_rem = k_rem - ca
        lax.fori_loop(0, 16, zero_hist, 0)
        n3, acc = refine_scan(n2, b2, 8, acc, True)

        b3, ca = select_bin(k_rem)
        k_rem = k_rem - ca
        _, acc = refine_scan(n3, b3, 0, acc, False)

        sum_gt = jnp.sum(acc)
        t_key = jnp.bitwise_or(
            jnp.left_shift(b0, 24),
            jnp.bitwise_or(jnp.left_shift(b1, 16),
                           jnp.bitwise_or(jnp.left_shift(b2, 8), b3)))

        tkv = jnp.full((16,), 1, jnp.int32) * t_key
        negk = tkv >= 0
        borig = jnp.where(negk, jnp.bitwise_not(tkv),
                          jnp.bitwise_xor(tkv, jnp.int32(MININT)))
        vtv = lax.bitcast_convert_type(borig, jnp.float32)
        ev = 1.0 + vtv
        kremf = lax.convert_element_type(k_rem, jnp.float32)
        tie_v = jnp.where(jnp.logical_and(iota == 0, k_rem > 0),
                          kremf * ev * ev, 0.0)
        hard_sum = sum_gt + jnp.sum(tie_v)

        ones_f = jnp.full((16,), 1.0, jnp.float32)
        nposf_v = ones_f * lax.convert_element_type(n_pos, jnp.float32)
        numf_v = ones_f * lax.convert_element_type(num, jnp.float32)
        loss_v = (jnp.float32(DELTA) * (ones_f * pos_sum) / nposf_v
                  + (ones_f * hard_sum) / numf_v)
        loss_acc = loss_acc + loss_v

    stage[...] = jnp.where(iota == 0, loss_acc, 0.0)
    out_off = pl.multiple_of(wid * 16, 16)
    pltpu.sync_copy(stage, out_hbm.at[pl.ds(out_off, 16)])


def kernel(logits, labels):
    out = _mmcl_sc(logits.reshape(-1), labels.reshape(-1))
    return jnp.sum(out) * jnp.float32(1.0 / N_ROWS)


# speedup vs baseline: 37.3976x; 1.2025x over previous
import functools

import jax
import jax.numpy as jnp
from jax import lax
from jax.experimental import pallas as pl
from jax.experimental.pallas import tpu as pltpu
from jax.experimental.pallas import tpu_sc as plsc

N_ROWS = 64
M_COLS = 100000
CH = 2000
NDC = M_COLS // (2 * CH)
VPC = CH // 16
UNROLL = 5
KEY_CAP = M_COLS + 16
DELTA = 5.0
INV_R = 100
MININT = -(2 ** 31)

_MESH = plsc.VectorSubcoreMesh(core_axis_name="c", subcore_axis_name="s")


@functools.partial(
    pl.kernel,
    mesh=_MESH,
    out_type=jax.ShapeDtypeStruct((512,), jnp.float32),
    compiler_params=pltpu.CompilerParams(needs_layout_passes=False),
    scratch_types=[
        pltpu.VMEM((CH,), jnp.float32),
        pltpu.VMEM((CH,), jnp.float32),
        pltpu.VMEM((CH,), jnp.int32),
        pltpu.VMEM((CH,), jnp.int32),
        pltpu.VMEM((KEY_CAP,), jnp.int32),
        pltpu.VMEM((256,), jnp.int32),
        pltpu.VMEM((16,), jnp.float32),
        pltpu.SemaphoreType.DMA,
        pltpu.SemaphoreType.DMA,
        pltpu.SemaphoreType.DMA,
        pltpu.SemaphoreType.DMA,
    ],
)
def _mmcl_sc(logits_hbm, labels_hbm, out_hbm, vbuf0, vbuf1, lbuf0, lbuf1,
             keys, hcnt, stage, sv0, sv1, sl0, sl1):
    wid = lax.axis_index("s") * 2 + lax.axis_index("c")
    iota = lax.iota(jnp.int32, 16)
    zeros_i = jnp.zeros((16,), jnp.int32)
    zeros_f = jnp.zeros((16,), jnp.float32)
    ones_i = jnp.ones((16,), jnp.int32)

    def zero_hist(g, carry):
        hcnt[pl.ds(g * 16, 16)] = zeros_i
        return carry

    def start_pair(base, vb, lb, sv, sl):
        pltpu.make_async_copy(logits_hbm.at[pl.ds(base, CH)], vb, sv).start()
        pltpu.make_async_copy(labels_hbm.at[pl.ds(base, CH)], lb, sl).start()

    def wait_pair(base, vb, lb, sv, sl):
        pltpu.make_async_copy(logits_hbm.at[pl.ds(base, CH)], vb, sv).wait()
        pltpu.make_async_copy(labels_hbm.at[pl.ds(base, CH)], lb, sl).wait()

    def chunk_compute(vb, lb, possum, offv):
        def vec_body(i, icarry):
            possum, offv = icarry
            for u in range(UNROLL):
                sl = pl.ds(i * (16 * UNROLL) + u * 16, 16)
                v = vb[sl]
                labv = lb[sl]
                isneg = labv == 0
                ispos = jnp.logical_not(isneg)
                d = 1.0 - v
                possum = possum + jnp.where(ispos, d * d, 0.0)
                bbits = lax.bitcast_convert_type(v, jnp.int32)
                sgn = jnp.right_shift(bbits, 31)
                key = jnp.bitwise_xor(bbits,
                                      jnp.bitwise_or(sgn, jnp.int32(MININT)))
                incl = plsc.cumsum(jnp.where(isneg, ones_i, 0))
                posn = offv + incl - 1
                plsc.store_scatter(keys, [posn], key, mask=isneg)
                bin0 = jnp.bitwise_and(jnp.right_shift(key, 24), 255)
                plsc.addupdate_scatter(hcnt, [bin0], ones_i, mask=isneg)
                offv = offv + plsc.all_reduce_population_count(isneg)
            return possum, offv

        return lax.fori_loop(0, VPC // UNROLL, vec_body, (possum, offv))

    def select_bin(k_rem):
        def sel_a(gi, carry):
            bcount, runs = carry
            g = 15 - gi
            c = hcnt[pl.ds(g * 16, 16)]
            incl = plsc.cumsum(lax.rev(c, (0,)))
            suff = lax.rev(incl, (0,)) + runs
            m = suff >= k_rem
            bcount = bcount + jnp.sum(jnp.where(m, ones_i, 0))
            runs = runs + jnp.sum(c)
            return bcount, runs

        bcount, _ = lax.fori_loop(0, 16, sel_a, (jnp.int32(0), jnp.int32(0)))
        b = bcount - 1

        def sel_b(g, ca):
            m = (g * 16 + iota) > b
            return ca + jnp.where(m, hcnt[pl.ds(g * 16, 16)], 0)

        ca = lax.fori_loop(0, 16, sel_b, zeros_i)
        return b, jnp.sum(ca)

    def refine_scan(n_cur, b_sel, sh, acc, with_hist):
        def body(i, c):
            off2, acc = c
            keyvs = [keys[pl.ds(i * (16 * UNROLL) + u * 16, 16)]
                     for u in range(UNROLL)]
            for u in range(UNROLL):
                base = i * (16 * UNROLL) + u * 16
                key = keyvs[u]
                valid = (base + iota) < n_cur
                binv = jnp.bitwise_and(jnp.right_shift(key, sh), 255)
                above = jnp.logical_and(valid, binv > b_sel)
                matched = jnp.logical_and(valid, binv == b_sel)
                negk = key >= 0
                borig = jnp.where(negk, jnp.bitwise_not(key),
                                  jnp.bitwise_xor(key, jnp.int32(MININT)))
                vdec = lax.bitcast_convert_type(borig, jnp.float32)
                e = 1.0 + vdec
                acc = acc + jnp.where(above, e * e, 0.0)
                incl = plsc.cumsum(jnp.where(matched, ones_i, 0))
                posn = off2 + incl - 1
                plsc.store_scatter(keys, [posn], key, mask=matched)
                if with_hist:
                    nbin = jnp.bitwise_and(jnp.right_shift(key, sh - 8), 255)
                    plsc.addupdate_scatter(hcnt, [nbin], ones_i, mask=matched)
                off2 = off2 + plsc.all_reduce_population_count(matched)
            return off2, acc

        niter = (n_cur + 16 * UNROLL - 1) // (16 * UNROLL)
        off2, acc = lax.fori_loop(0, niter, body, (zeros_i, acc))
        return jnp.max(off2), acc

    loss_acc = zeros_f
    for r in range(2):
        row = wid * 2 + r
        base0 = pl.multiple_of(row * M_COLS, 16)
        lax.fori_loop(0, 16, zero_hist, 0)

        start_pair(base0, vbuf0, lbuf0, sv0, sl0)

        def jbody(j, carry, base0=base0):
            possum, offv = carry
            base = pl.multiple_of(base0 + j * 2 * CH, 16)
            start_pair(base + CH, vbuf1, lbuf1, sv1, sl1)
            wait_pair(base, vbuf0, lbuf0, sv0, sl0)
            possum, offv = chunk_compute(vbuf0, lbuf0, possum, offv)

            @pl.when(j < NDC - 1)
            def _():
                start_pair(base + 2 * CH, vbuf0, lbuf0, sv0, sl0)

            wait_pair(base + CH, vbuf1, lbuf1, sv1, sl1)
            possum, offv = chunk_compute(vbuf1, lbuf1, possum, offv)
            return possum, offv

        possum, offv = lax.fori_loop(0, NDC, jbody, (zeros_f, zeros_i))

        pos_sum = jnp.sum(possum)
        n_neg = jnp.max(offv)
        n_pos = M_COLS - n_neg
        num = n_neg // INV_R

        k_rem = num
        acc = zeros_f
        b0, ca = select_bin(k_rem)
        k_rem = k_rem - ca
        lax.fori_loop(0, 16, zero_hist, 0)
        n1, acc = refine_scan(n_neg, b0, 24, acc, True)

        b1, ca = select_bin(k_rem)
        k_rem = k_rem - ca
        lax.fori_loop(0, 16, zero_hist, 0)
        n2, acc = refine_scan(n1, b1, 16, acc, True)

        b2, ca = select_bin(k_rem)
        k_rem = k_rem - ca
        lax.fori_loop(0, 16, zero_hist, 0)
        n3, acc = refine_scan(n2, b2, 8, acc, True)

        b3, ca = select_bin(k_rem)
        k_rem = k_rem - ca
        _, acc = refine_scan(n3, b3, 0, acc, False)

        sum_gt = jnp.sum(acc)
        t_key = jnp.bitwise_or(
            jnp.left_shift(b0, 24),
            jnp.bitwise_or(jnp.left_shift(b1, 16),
                           jnp.bitwise_or(jnp.left_shift(b2, 8), b3)))

        tkv = jnp.full((16,), 1, jnp.int32) * t_key
        negk = tkv >= 0
        borig = jnp.where(negk, jnp.bitwise_not(tkv),
                          jnp.bitwise_xor(tkv, jnp.int32(MININT)))
        vtv = lax.bitcast_convert_type(borig, jnp.float32)
        ev = 1.0 + vtv
        kremf = lax.convert_element_type(k_rem, jnp.float32)
        tie_v = jnp.where(jnp.logical_and(iota == 0, k_rem > 0),
                          kremf * ev * ev, 0.0)
        hard_sum = sum_gt + jnp.sum(tie_v)

        ones_f = jnp.full((16,), 1.0, jnp.float32)
        nposf_v = ones_f * lax.convert_element_type(n_pos, jnp.float32)
        numf_v = ones_f * lax.convert_element_type(num, jnp.float32)
        loss_v = (jnp.float32(DELTA) * (ones_f * pos_sum) / nposf_v
                  + (ones_f * hard_sum) / numf_v)
        loss_acc = loss_acc + loss_v

    stage[...] = jnp.where(iota == 0, loss_acc, 0.0)
    out_off = pl.multiple_of(wid * 16, 16)
    pltpu.sync_copy(stage, out_hbm.at[pl.ds(out_off, 16)])


def kernel(logits, labels):
    out = _mmcl_sc(logits.reshape(-1), labels.reshape(-1))
    return jnp.sum(out) * jnp.float32(1.0 / N_ROWS)
